# baseline (speedup 1.0000x reference)
import functools
import jax
import jax.numpy as jnp
from jax import lax
from jax.experimental import pallas as pl
from jax.experimental.pallas import tpu as pltpu
from jax.experimental.pallas import tpu_sc as plsc

N, C, HIN, WIN = 4, 96, 192, 192
HOUT, WOUT = 384, 384
NP = N * C
NIDX = HIN * WIN
PLANE = HOUT * WOUT

NSEG = 2
SEG = PLANE // NSEG
TASKS = NP * NSEG
NWORK = 32
TPW = TASKS // NWORK
CH = 9216
NCHUNK = NIDX // CH
L = 16


def _unpool_body(x_hbm, idx_hbm, out_hbm, seg_buf, idx_v, val_v, sem_a, sem_b):
    wid = lax.axis_index("s") * 2 + lax.axis_index("c")

    zeros = jnp.zeros((L,), jnp.float32)
    sems = (sem_a, sem_b)

    @plsc.parallel_loop(0, SEG, L, unroll=8)
    def _(i):
        seg_buf[pl.ds(i, L)] = zeros

    def task_body(t, carry):
        task = wid * TPW + t
        plane = task // NSEG
        seg = task % NSEG
        base = (seg * SEG).astype(jnp.int32)

        descs = [
            pltpu.async_copy(idx_hbm.at[plane, pl.ds(0, CH)], idx_v.at[0],
                             sems[0]),
            pltpu.async_copy(x_hbm.at[plane, pl.ds(0, CH)], val_v.at[0],
                             sems[0]),
        ]

        for k in range(NCHUNK):
            b = k % 2
            descs[0].wait()
            descs[1].wait()
            if k + 1 < NCHUNK:
                nb = (k + 1) % 2
                descs = [
                    pltpu.async_copy(
                        idx_hbm.at[plane, pl.ds((k + 1) * CH, CH)],
                        idx_v.at[nb], sems[nb]),
                    pltpu.async_copy(
                        x_hbm.at[plane, pl.ds((k + 1) * CH, CH)],
                        val_v.at[nb], sems[nb]),
                ]

            @plsc.parallel_loop(0, CH, L, unroll=8)
            def _(j, b=b):
                iv = idx_v[b, pl.ds(j, L)]
                vv = val_v[b, pl.ds(j, L)]
                loc = iv - base
                m = plsc.bitcast(loc, jnp.uint32) < jnp.uint32(SEG)
                plsc.store_scatter(seg_buf, [loc], vv, mask=m)

        H = SEG // 2
        d0 = pltpu.async_copy(seg_buf.at[pl.ds(0, H)],
                              out_hbm.at[plane, pl.ds(base, H)], sem_a)
        d1 = pltpu.async_copy(seg_buf.at[pl.ds(H, H)],
                              out_hbm.at[plane, pl.ds(base + H, H)], sem_b)
        d0.wait()

        @plsc.parallel_loop(0, H, L, unroll=8)
        def _(i):
            seg_buf[pl.ds(i, L)] = zeros

        d1.wait()

        @plsc.parallel_loop(H, SEG, L, unroll=8)
        def _(i):
            seg_buf[pl.ds(i, L)] = zeros

        return carry

    lax.fori_loop(0, TPW, task_body, 0)


@jax.jit
def _unpool(x2d, idx2d):
    mesh = plsc.VectorSubcoreMesh(core_axis_name="c", subcore_axis_name="s")
    return pl.kernel(
        _unpool_body,
        out_type=jax.ShapeDtypeStruct((NP, PLANE), jnp.float32),
        mesh=mesh,
        compiler_params=pltpu.CompilerParams(
            needs_layout_passes=False, use_tc_tiling_on_sc=False),
        scratch_types=[
            pltpu.VMEM((SEG,), jnp.float32),
            pltpu.VMEM((2, CH), jnp.int32),
            pltpu.VMEM((2, CH), jnp.float32),
            pltpu.SemaphoreType.DMA,
            pltpu.SemaphoreType.DMA,
        ],
    )(x2d, idx2d)


def kernel(x, indices, output_size):
    x2d = x.reshape(NP, NIDX)
    idx2d = indices.reshape(NP, NIDX)
    out = _unpool(x2d, idx2d)
    return out.reshape(N, C, HOUT, WOUT)


# speedup vs baseline: 1.0589x; 1.0589x over previous
import functools
import jax
import jax.numpy as jnp
from jax import lax
from jax.experimental import pallas as pl
from jax.experimental.pallas import tpu as pltpu
from jax.experimental.pallas import tpu_sc as plsc

N, C, HIN, WIN = 4, 96, 192, 192
HOUT, WOUT = 384, 384
NP = N * C
NIDX = HIN * WIN
PLANE = HOUT * WOUT

NSEG = 2
SEG = PLANE // NSEG
TASKS = NP * NSEG
NWORK = 32
TPW = TASKS // NWORK
CH = 9216
NCHUNK = NIDX // CH
L = 16


def _unpool_body(x_hbm, idx_hbm, out_hbm, seg_buf, idx_v, val_v, sem_a, sem_b):
    wid = lax.axis_index("s") * 2 + lax.axis_index("c")

    zeros = jnp.zeros((L,), jnp.float32)
    sems = (sem_a, sem_b)

    def task_body(t, carry):
        task = wid * TPW + t
        plane = task // NSEG
        seg = task % NSEG
        base = (seg * SEG).astype(jnp.int32)

        descs = [
            pltpu.async_copy(idx_hbm.at[plane, pl.ds(0, CH)], idx_v.at[0],
                             sems[0]),
            pltpu.async_copy(x_hbm.at[plane, pl.ds(0, CH)], val_v.at[0],
                             sems[0]),
        ]

        @plsc.parallel_loop(0, SEG, L, unroll=8)
        def _(i):
            seg_buf[pl.ds(i, L)] = zeros

        for k in range(NCHUNK):
            b = k % 2
            descs[0].wait()
            descs[1].wait()
            if k + 1 < NCHUNK:
                nb = (k + 1) % 2
                descs = [
                    pltpu.async_copy(
                        idx_hbm.at[plane, pl.ds((k + 1) * CH, CH)],
                        idx_v.at[nb], sems[nb]),
                    pltpu.async_copy(
                        x_hbm.at[plane, pl.ds((k + 1) * CH, CH)],
                        val_v.at[nb], sems[nb]),
                ]

            @plsc.parallel_loop(0, CH, L, unroll=16)
            def _(j, b=b):
                iv = idx_v[b, pl.ds(j, L)]
                vv = val_v[b, pl.ds(j, L)]
                loc = iv - base
                m = plsc.bitcast(loc, jnp.uint32) < jnp.uint32(SEG)
                plsc.store_scatter(seg_buf, [loc], vv, mask=m)

        pltpu.sync_copy(seg_buf, out_hbm.at[plane, pl.ds(base, SEG)])
        return carry

    lax.fori_loop(0, TPW, task_body, 0)


@jax.jit
def _unpool(x2d, idx2d):
    mesh = plsc.VectorSubcoreMesh(core_axis_name="c", subcore_axis_name="s")
    return pl.kernel(
        _unpool_body,
        out_type=jax.ShapeDtypeStruct((NP, PLANE), jnp.float32),
        mesh=mesh,
        compiler_params=pltpu.CompilerParams(
            needs_layout_passes=False, use_tc_tiling_on_sc=False),
        scratch_types=[
            pltpu.VMEM((SEG,), jnp.float32),
            pltpu.VMEM((2, CH), jnp.int32),
            pltpu.VMEM((2, CH), jnp.float32),
            pltpu.SemaphoreType.DMA,
            pltpu.SemaphoreType.DMA,
        ],
    )(x2d, idx2d)


def kernel(x, indices, output_size):
    x2d = x.reshape(NP, NIDX)
    idx2d = indices.reshape(NP, NIDX)
    out = _unpool(x2d, idx2d)
    return out.reshape(N, C, HOUT, WOUT)


# speedup vs baseline: 1.0904x; 1.0298x over previous
import functools
import jax
import jax.numpy as jnp
from jax import lax
from jax.experimental import pallas as pl
from jax.experimental.pallas import tpu as pltpu
from jax.experimental.pallas import tpu_sc as plsc

N, C, HIN, WIN = 4, 96, 192, 192
HOUT, WOUT = 384, 384
NP = N * C
NIDX = HIN * WIN
PLANE = HOUT * WOUT

NSEG = 2
SEG = PLANE // NSEG
TASKS = NP * NSEG
NWORK = 32
TPW = TASKS // NWORK
CH = 12288
NCHUNK = NIDX // CH
L = 16


def _unpool_body(x_hbm, idx_hbm, out_hbm, seg_buf, idx_v, val_v, sem_a, sem_b):
    wid = lax.axis_index("s") * 2 + lax.axis_index("c")

    zeros = jnp.zeros((L,), jnp.float32)
    sems = (sem_a, sem_b)

    def task_body(t, carry):
        task = wid * TPW + t
        plane = task // NSEG
        seg = task % NSEG
        base = (seg * SEG).astype(jnp.int32)

        descs = [
            pltpu.async_copy(idx_hbm.at[plane, pl.ds(0, CH)], idx_v.at[0],
                             sems[0]),
            pltpu.async_copy(x_hbm.at[plane, pl.ds(0, CH)], val_v.at[0],
                             sems[0]),
        ]

        @plsc.parallel_loop(0, SEG, L, unroll=8)
        def _(i):
            seg_buf[pl.ds(i, L)] = zeros

        for k in range(NCHUNK):
            b = k % 2
            descs[0].wait()
            descs[1].wait()
            if k + 1 < NCHUNK:
                nb = (k + 1) % 2
                descs = [
                    pltpu.async_copy(
                        idx_hbm.at[plane, pl.ds((k + 1) * CH, CH)],
                        idx_v.at[nb], sems[nb]),
                    pltpu.async_copy(
                        x_hbm.at[plane, pl.ds((k + 1) * CH, CH)],
                        val_v.at[nb], sems[nb]),
                ]

            @plsc.parallel_loop(0, CH, L, unroll=16)
            def _(j, b=b):
                iv = idx_v[b, pl.ds(j, L)]
                vv = val_v[b, pl.ds(j, L)]
                loc = iv - base
                m = plsc.bitcast(loc, jnp.uint32) < jnp.uint32(SEG)
                plsc.store_scatter(seg_buf, [loc], vv, mask=m)

        pltpu.sync_copy(seg_buf, out_hbm.at[plane, pl.ds(base, SEG)])
        return carry

    lax.fori_loop(0, TPW, task_body, 0)


@jax.jit
def _unpool(x2d, idx2d):
    mesh = plsc.VectorSubcoreMesh(core_axis_name="c", subcore_axis_name="s")
    return pl.kernel(
        _unpool_body,
        out_type=jax.ShapeDtypeStruct((NP, PLANE), jnp.float32),
        mesh=mesh,
        compiler_params=pltpu.CompilerParams(
            needs_layout_passes=False, use_tc_tiling_on_sc=False),
        scratch_types=[
            pltpu.VMEM((SEG,), jnp.float32),
            pltpu.VMEM((2, CH), jnp.int32),
            pltpu.VMEM((2, CH), jnp.float32),
            pltpu.SemaphoreType.DMA,
            pltpu.SemaphoreType.DMA,
        ],
    )(x2d, idx2d)


def kernel(x, indices, output_size):
    x2d = x.reshape(NP, NIDX)
    idx2d = indices.reshape(NP, NIDX)
    out = _unpool(x2d, idx2d)
    return out.reshape(N, C, HOUT, WOUT)


# speedup vs baseline: 1.0960x; 1.0051x over previous
import functools
import jax
import jax.numpy as jnp
from jax import lax
from jax.experimental import pallas as pl
from jax.experimental.pallas import tpu as pltpu
from jax.experimental.pallas import tpu_sc as plsc

N, C, HIN, WIN = 4, 96, 192, 192
HOUT, WOUT = 384, 384
NP = N * C
NIDX = HIN * WIN
PLANE = HOUT * WOUT

NSEG = 2
SEG = PLANE // NSEG
TASKS = NP * NSEG
NWORK = 32
TPW = TASKS // NWORK
CH = 12288
NCHUNK = NIDX // CH
L = 16


def _unpool_body(x_hbm, idx_hbm, out_hbm, seg_buf, idx_v, val_v, sem_a, sem_b):
    wid = lax.axis_index("s") * 2 + lax.axis_index("c")

    zeros = jnp.zeros((L,), jnp.float32)
    sems = (sem_a, sem_b)

    def task_body(t, carry):
        task = wid * TPW + t
        plane = task // NSEG
        seg = task % NSEG
        base = (seg * SEG).astype(jnp.int32)

        descs = [
            pltpu.async_copy(idx_hbm.at[plane, pl.ds(0, CH)], idx_v.at[0],
                             sems[0]),
            pltpu.async_copy(x_hbm.at[plane, pl.ds(0, CH)], val_v.at[0],
                             sems[0]),
        ]

        @plsc.parallel_loop(0, SEG, L, unroll=16)
        def _(i):
            seg_buf[pl.ds(i, L)] = zeros

        for k in range(NCHUNK):
            b = k % 2
            descs[0].wait()
            descs[1].wait()
            if k + 1 < NCHUNK:
                nb = (k + 1) % 2
                descs = [
                    pltpu.async_copy(
                        idx_hbm.at[plane, pl.ds((k + 1) * CH, CH)],
                        idx_v.at[nb], sems[nb]),
                    pltpu.async_copy(
                        x_hbm.at[plane, pl.ds((k + 1) * CH, CH)],
                        val_v.at[nb], sems[nb]),
                ]

            @plsc.parallel_loop(0, CH, L, unroll=16)
            def _(j, b=b):
                iv = idx_v[b, pl.ds(j, L)]
                vv = val_v[b, pl.ds(j, L)]
                loc = iv - base
                m = plsc.bitcast(loc, jnp.uint32) < jnp.uint32(SEG)
                plsc.store_scatter(seg_buf, [loc], vv, mask=m)

        pltpu.sync_copy(seg_buf, out_hbm.at[plane, pl.ds(base, SEG)])
        return carry

    lax.fori_loop(0, TPW, task_body, 0)


@jax.jit
def _unpool(x2d, idx2d):
    mesh = plsc.VectorSubcoreMesh(core_axis_name="c", subcore_axis_name="s")
    return pl.kernel(
        _unpool_body,
        out_type=jax.ShapeDtypeStruct((NP, PLANE), jnp.float32),
        mesh=mesh,
        compiler_params=pltpu.CompilerParams(
            needs_layout_passes=False, use_tc_tiling_on_sc=False),
        scratch_types=[
            pltpu.VMEM((SEG,), jnp.float32),
            pltpu.VMEM((2, CH), jnp.int32),
            pltpu.VMEM((2, CH), jnp.float32),
            pltpu.SemaphoreType.DMA,
            pltpu.SemaphoreType.DMA,
        ],
    )(x2d, idx2d)


def kernel(x, indices, output_size):
    x2d = x.reshape(NP, NIDX)
    idx2d = indices.reshape(NP, NIDX)
    out = _unpool(x2d, idx2d)
    return out.reshape(N, C, HOUT, WOUT)
